# baseline (speedup 1.0000x reference)
import functools

import jax
import jax.numpy as jnp
from jax import lax
from jax.experimental import pallas as pl
from jax.experimental.pallas import tpu as pltpu
from jax.experimental.pallas import tpu_sc as plsc

H = 224
W = 224
HW = H * W
HALF = HW // 2
DBUF = HALF + 32
NSUB = 6250
NPAD = 6256


def _project_body(vt_ref, rot_ref, trans_ref, intr_ref,
                  p0_ref, p1_ref, dep_ref):
    def rb(s):
        return s.astype(jnp.bfloat16).astype(jnp.float32)

    x = vt_ref[0, 0].astype(jnp.float32)
    y = vt_ref[0, 1].astype(jnp.float32)
    z = vt_ref[0, 2].astype(jnp.float32)
    r00 = rot_ref[0, 0, 0]
    r01 = rot_ref[0, 0, 1]
    r02 = rot_ref[0, 0, 2]
    r10 = rot_ref[0, 1, 0]
    r11 = rot_ref[0, 1, 1]
    r12 = rot_ref[0, 1, 2]
    r20 = rot_ref[0, 2, 0]
    r21 = rot_ref[0, 2, 1]
    r22 = rot_ref[0, 2, 2]
    tx = trans_ref[0, 0, 0]
    ty = trans_ref[0, 0, 1]
    tz = trans_ref[0, 0, 2]
    fx = intr_ref[0, 0, 0]
    fy = intr_ref[0, 1, 1]
    cx = intr_ref[0, 0, 2]
    cy = intr_ref[0, 1, 2]

    X = x * rb(r00) + y * rb(r01) + z * rb(r02) + tx
    Y = x * rb(r10) + y * rb(r11) + z * rb(r12) + ty
    Z = x * rb(r20) + y * rb(r21) + z * rb(r22) + tz
    Zs = Z + 1e-8
    u = fx * (X / Zs) + cx
    v = fy * (Y / Zs) + cy
    u_i = u.astype(jnp.int32)
    v_i = v.astype(jnp.int32)
    valid = (u_i >= 0) & (u_i < W) & (v_i >= 0) & (v_i < H)
    pix = jnp.where(valid, v_i * W + u_i, HW)
    p0 = jnp.minimum(pix, HALF)
    p1u = pix - HALF
    p1 = jnp.where(p1u < 0, HALF, jnp.minimum(p1u, HALF))

    p0_ref[:, 0, :] = jnp.full((8, NPAD), HALF, jnp.int32)
    p1_ref[:, 0, :] = jnp.full((8, NPAD), HALF, jnp.int32)
    p0_ref[:, 0, 0:NSUB] = p0
    p1_ref[:, 0, 0:NSUB] = p1
    dep_ref[:, 0, 0:NSUB] = Z


def _tc_project(verts_t, rotation, translation, intrinsics):
    B = rotation.shape[0]
    out_shape = (
        jax.ShapeDtypeStruct((B * 8, 1, NPAD), jnp.int32),
        jax.ShapeDtypeStruct((B * 8, 1, NPAD), jnp.int32),
        jax.ShapeDtypeStruct((B * 8, 1, NPAD), jnp.float32),
    )
    return pl.pallas_call(
        _project_body,
        grid=(B,),
        in_specs=[
            pl.BlockSpec((1, 3, 8, NSUB), lambda b: (b, 0, 0, 0)),
            pl.BlockSpec((1, 3, 3), lambda b: (b, 0, 0),
                         memory_space=pltpu.SMEM),
            pl.BlockSpec((1, 1, 3), lambda b: (b, 0, 0),
                         memory_space=pltpu.SMEM),
            pl.BlockSpec((1, 3, 3), lambda b: (b, 0, 0),
                         memory_space=pltpu.SMEM),
        ],
        out_specs=[
            pl.BlockSpec((8, 1, NPAD), lambda b: (b, 0, 0)),
            pl.BlockSpec((8, 1, NPAD), lambda b: (b, 0, 0)),
            pl.BlockSpec((8, 1, NPAD), lambda b: (b, 0, 0)),
        ],
        out_shape=out_shape,
    )(verts_t, rotation, translation, intrinsics)


def _sc_scatter(p0, p1, dep, B):
    mesh = plsc.VectorSubcoreMesh(core_axis_name="c", subcore_axis_name="s")

    @functools.partial(
        pl.kernel,
        mesh=mesh,
        out_type=jax.ShapeDtypeStruct((B * HW,), jnp.float32),
        compiler_params=pltpu.CompilerParams(needs_layout_passes=False),
        scratch_types=[
            pltpu.VMEM((DBUF,), jnp.float32),
            pltpu.VMEM((8, 1, NPAD), jnp.int32),
            pltpu.VMEM((8, 1, NPAD), jnp.float32),
            pltpu.SemaphoreType.DMA,
        ],
    )
    def scatter_kernel(p0_hbm, p1_hbm, dep_hbm, out_hbm,
                       dbuf, pixv, depv, sem):
        cid = lax.axis_index("c")
        sid = lax.axis_index("s")
        b = sid
        lo = cid * HALF

        @pl.when(cid == 0)
        def _():
            pltpu.async_copy(p0_hbm.at[pl.ds(b * 8, 8)], pixv, sem)

        @pl.when(cid == 1)
        def _():
            pltpu.async_copy(p1_hbm.at[pl.ds(b * 8, 8)], pixv, sem)

        d2 = pltpu.async_copy(dep_hbm.at[pl.ds(b * 8, 8)], depv, sem)

        zeros = jnp.zeros((16,), jnp.float32)

        def zero_body(j, carry):
            dbuf[pl.ds(j * 16, 16)] = zeros
            return carry

        lax.fori_loop(0, DBUF // 16, zero_body, 0, unroll=8)
        d2.wait()
        d2.wait()

        for r in range(8):
            def vec_body(i, carry, r=r):
                p = pixv[r, 0, pl.ds(i * 16, 16)]
                d = depv[r, 0, pl.ds(i * 16, 16)]
                plsc.store_scatter(dbuf, [p], d)
                return carry

            lax.fori_loop(0, NPAD // 16, vec_body, 0, unroll=4)

        out_off = pl.multiple_of(b * HW + lo, 8)
        pltpu.sync_copy(dbuf.at[pl.ds(0, HALF)],
                        out_hbm.at[pl.ds(out_off, HALF)])

    return scatter_kernel(p0, p1, dep)


def kernel(vertices, rotation, translation, camera_intrinsics):
    B, N, _ = vertices.shape
    verts_t = jnp.swapaxes(vertices.astype(jnp.bfloat16),
                           1, 2).reshape(B, 3, 8, N // 8)
    p0, p1, dep = _tc_project(verts_t, rotation,
                              translation.reshape(B, 1, 3),
                              camera_intrinsics)
    flat = _sc_scatter(p0, p1, dep, B)
    return flat.reshape(B, 1, H, W)


# speedup vs baseline: 1.0038x; 1.0038x over previous
import functools

import jax
import jax.numpy as jnp
from jax import lax
from jax.experimental import pallas as pl
from jax.experimental.pallas import tpu as pltpu
from jax.experimental.pallas import tpu_sc as plsc

H = 224
W = 224
HW = H * W
HALF = HW // 2
DBUF = HALF + 32
NSUB = 6250
NPAD = 6256


def _project_body(vt_ref, rot_ref, trans_ref, intr_ref,
                  p0_ref, p1_ref, dep_ref):
    def rb(s):
        return s.astype(jnp.bfloat16).astype(jnp.float32)

    x = vt_ref[0, 0].astype(jnp.float32)
    y = vt_ref[0, 1].astype(jnp.float32)
    z = vt_ref[0, 2].astype(jnp.float32)
    r00 = rot_ref[0, 0, 0]
    r01 = rot_ref[0, 0, 1]
    r02 = rot_ref[0, 0, 2]
    r10 = rot_ref[0, 1, 0]
    r11 = rot_ref[0, 1, 1]
    r12 = rot_ref[0, 1, 2]
    r20 = rot_ref[0, 2, 0]
    r21 = rot_ref[0, 2, 1]
    r22 = rot_ref[0, 2, 2]
    tx = trans_ref[0, 0, 0]
    ty = trans_ref[0, 0, 1]
    tz = trans_ref[0, 0, 2]
    fx = intr_ref[0, 0, 0]
    fy = intr_ref[0, 1, 1]
    cx = intr_ref[0, 0, 2]
    cy = intr_ref[0, 1, 2]

    X = x * rb(r00) + y * rb(r01) + z * rb(r02) + tx
    Y = x * rb(r10) + y * rb(r11) + z * rb(r12) + ty
    Z = x * rb(r20) + y * rb(r21) + z * rb(r22) + tz
    Zs = Z + 1e-8
    u = fx * (X / Zs) + cx
    v = fy * (Y / Zs) + cy
    u_i = u.astype(jnp.int32)
    v_i = v.astype(jnp.int32)
    valid = (u_i >= 0) & (u_i < W) & (v_i >= 0) & (v_i < H)
    pix = jnp.where(valid, v_i * W + u_i, HW)
    p0 = jnp.minimum(pix, HALF)
    p1u = pix - HALF
    p1 = jnp.where(p1u < 0, HALF, jnp.minimum(p1u, HALF))

    p0_ref[:, 0, :] = jnp.full((8, NPAD), HALF, jnp.int32)
    p1_ref[:, 0, :] = jnp.full((8, NPAD), HALF, jnp.int32)
    p0_ref[:, 0, 0:NSUB] = p0
    p1_ref[:, 0, 0:NSUB] = p1
    dep_ref[:, 0, 0:NSUB] = Z


def _tc_project(verts_t, rotation, translation, intrinsics):
    B = rotation.shape[0]
    out_shape = (
        jax.ShapeDtypeStruct((B * 8, 1, NPAD), jnp.int32),
        jax.ShapeDtypeStruct((B * 8, 1, NPAD), jnp.int32),
        jax.ShapeDtypeStruct((B * 8, 1, NPAD), jnp.float32),
    )
    return pl.pallas_call(
        _project_body,
        grid=(B,),
        in_specs=[
            pl.BlockSpec((1, 3, 8, NSUB), lambda b: (b, 0, 0, 0)),
            pl.BlockSpec((1, 3, 3), lambda b: (b, 0, 0),
                         memory_space=pltpu.SMEM),
            pl.BlockSpec((1, 1, 3), lambda b: (b, 0, 0),
                         memory_space=pltpu.SMEM),
            pl.BlockSpec((1, 3, 3), lambda b: (b, 0, 0),
                         memory_space=pltpu.SMEM),
        ],
        out_specs=[
            pl.BlockSpec((8, 1, NPAD), lambda b: (b, 0, 0)),
            pl.BlockSpec((8, 1, NPAD), lambda b: (b, 0, 0)),
            pl.BlockSpec((8, 1, NPAD), lambda b: (b, 0, 0)),
        ],
        out_shape=out_shape,
    )(verts_t, rotation, translation, intrinsics)


def _sc_scatter(p0, p1, dep, B):
    mesh = plsc.VectorSubcoreMesh(core_axis_name="c", subcore_axis_name="s")

    @functools.partial(
        pl.kernel,
        mesh=mesh,
        out_type=jax.ShapeDtypeStruct((B * HW,), jnp.float32),
        compiler_params=pltpu.CompilerParams(needs_layout_passes=False),
        scratch_types=[
            pltpu.VMEM((DBUF,), jnp.float32),
            pltpu.VMEM((8, NPAD), jnp.int32),
            pltpu.VMEM((8, NPAD), jnp.float32),
            pltpu.SemaphoreType.DMA,
        ],
    )
    def scatter_kernel(p0_hbm, p1_hbm, dep_hbm, out_hbm,
                       dbuf, pixv, depv, sem):
        cid = lax.axis_index("c")
        sid = lax.axis_index("s")
        b = sid
        lo = cid * HALF

        @pl.when(cid == 0)
        def _():
            pltpu.async_copy(p0_hbm.at[pl.ds(b * 8, 8), 0], pixv, sem)

        @pl.when(cid == 1)
        def _():
            pltpu.async_copy(p1_hbm.at[pl.ds(b * 8, 8), 0], pixv, sem)

        d2 = pltpu.async_copy(dep_hbm.at[pl.ds(b * 8, 8), 0], depv, sem)

        zeros = jnp.zeros((16,), jnp.float32)

        def zero_body(j, carry):
            dbuf[pl.ds(j * 16, 16)] = zeros
            return carry

        lax.fori_loop(0, DBUF // 16, zero_body, 0, unroll=8)
        d2.wait()
        d2.wait()

        for r in range(8):
            def vec_body(i, carry, r=r):
                p = pixv[r, pl.ds(i * 16, 16)]
                d = depv[r, pl.ds(i * 16, 16)]
                plsc.store_scatter(dbuf, [p], d)
                return carry

            lax.fori_loop(0, NPAD // 16, vec_body, 0, unroll=17)

        out_off = pl.multiple_of(b * HW + lo, 8)
        pltpu.sync_copy(dbuf.at[pl.ds(0, HALF)],
                        out_hbm.at[pl.ds(out_off, HALF)])

    return scatter_kernel(p0, p1, dep)


def kernel(vertices, rotation, translation, camera_intrinsics):
    B, N, _ = vertices.shape
    verts_t = jnp.swapaxes(vertices.astype(jnp.bfloat16),
                           1, 2).reshape(B, 3, 8, N // 8)
    p0, p1, dep = _tc_project(verts_t, rotation,
                              translation.reshape(B, 1, 3),
                              camera_intrinsics)
    flat = _sc_scatter(p0, p1, dep, B)
    return flat.reshape(B, 1, H, W)
